# baseline (speedup 1.0000x reference)
import functools

import jax
import jax.numpy as jnp
from jax import lax
from jax.experimental import pallas as pl
from jax.experimental.pallas import tpu as pltpu
from jax.experimental.pallas import tpu_sc as plsc

EMBED = 64
DEMO = 16
H1 = 128
H2 = 64
NC, NS = 2, 16
NW = NC * NS
BM = 2048


def _sc_gather(user_idx, song_idx, user_table, song_table):
    batch = user_idx.shape[0]
    b_per_w = batch // NW
    mesh = plsc.VectorSubcoreMesh(core_axis_name="c", subcore_axis_name="s")
    out_type = (
        jax.ShapeDtypeStruct((batch, EMBED), jnp.float32),
        jax.ShapeDtypeStruct((batch, EMBED), jnp.float32),
    )

    @functools.partial(
        pl.kernel,
        mesh=mesh,
        out_type=out_type,
        compiler_params=pltpu.CompilerParams(use_tc_tiling_on_sc=False),
        scratch_types=[
            pltpu.VMEM((b_per_w,), jnp.int32),
            pltpu.VMEM((b_per_w,), jnp.int32),
            pltpu.VMEM((b_per_w, EMBED), jnp.float32),
            pltpu.VMEM((b_per_w, EMBED), jnp.float32),
            pltpu.SemaphoreType.DMA,
            pltpu.SemaphoreType.DMA,
        ],
    )
    def gather_kernel(ut_hbm, st_hbm, ui_hbm, si_hbm, uo_hbm, so_hbm,
                      ui_v, si_v, ur_v, sr_v, sem_u, sem_s):
        wid = lax.axis_index("s") * NC + lax.axis_index("c")
        base = wid * b_per_w
        pltpu.sync_copy(ui_hbm.at[pl.ds(base, b_per_w)], ui_v)
        pltpu.sync_copy(si_hbm.at[pl.ds(base, b_per_w)], si_v)
        cu = pltpu.async_copy(ut_hbm.at[ui_v], ur_v, sem_u)
        cs = pltpu.async_copy(st_hbm.at[si_v], sr_v, sem_s)
        cu.wait()
        pltpu.sync_copy(ur_v, uo_hbm.at[pl.ds(base, b_per_w)])
        cs.wait()
        pltpu.sync_copy(sr_v, so_hbm.at[pl.ds(base, b_per_w)])

    return gather_kernel(user_table, song_table, user_idx, song_idx)


def _mlp_body(u_ref, s_ref, d_ref, w1u_ref, w1s_ref, w1d_ref, b1_ref,
              w2_ref, b2_ref, w3_ref, b3_ref, o_ref):
    d_val = d_ref[...]
    d_val = jnp.where(jnp.isnan(d_val), jnp.float32(0.0), d_val)
    h = jnp.dot(u_ref[...], w1u_ref[...], preferred_element_type=jnp.float32)
    h = h + jnp.dot(s_ref[...], w1s_ref[...], preferred_element_type=jnp.float32)
    h = h + jnp.dot(d_val, w1d_ref[...], preferred_element_type=jnp.float32)
    h = jnp.maximum(h + b1_ref[...], 0.0)
    h2 = jnp.dot(h, w2_ref[...], preferred_element_type=jnp.float32)
    h2 = jnp.maximum(h2 + b2_ref[...], 0.0)
    logit = jnp.dot(h2, w3_ref[...], preferred_element_type=jnp.float32)
    o_ref[...] = jax.nn.sigmoid(logit + b3_ref[...])


def _tc_mlp(u_emb, s_emb, demo, W1, b1, W2, b2, W3, b3):
    batch = u_emb.shape[0]
    w1u = W1[:EMBED]
    w1s = W1[EMBED:2 * EMBED]
    w1d = W1[2 * EMBED:]
    out = pl.pallas_call(
        _mlp_body,
        grid=(batch // BM,),
        in_specs=[
            pl.BlockSpec((BM, EMBED), lambda i: (i, 0)),
            pl.BlockSpec((BM, EMBED), lambda i: (i, 0)),
            pl.BlockSpec((BM, DEMO), lambda i: (i, 0)),
            pl.BlockSpec((EMBED, H1), lambda i: (0, 0)),
            pl.BlockSpec((EMBED, H1), lambda i: (0, 0)),
            pl.BlockSpec((DEMO, H1), lambda i: (0, 0)),
            pl.BlockSpec((1, H1), lambda i: (0, 0)),
            pl.BlockSpec((H1, H2), lambda i: (0, 0)),
            pl.BlockSpec((1, H2), lambda i: (0, 0)),
            pl.BlockSpec((H2, 1), lambda i: (0, 0)),
            pl.BlockSpec((1, 1), lambda i: (0, 0)),
        ],
        out_specs=pl.BlockSpec((BM, 1), lambda i: (i, 0)),
        out_shape=jax.ShapeDtypeStruct((batch, 1), jnp.float32),
    )(u_emb, s_emb, demo, w1u, w1s, w1d, b1.reshape(1, H1), W2,
      b2.reshape(1, H2), W3, b3.reshape(1, 1))
    return out


def kernel(user_input, song_input, demographic_input, user_table, song_table,
           W1, b1, W2, b2, W3, b3):
    u_emb, s_emb = _sc_gather(user_input, song_input, user_table, song_table)
    out = _tc_mlp(u_emb, s_emb, demographic_input, W1, b1, W2, b2, W3, b3)
    return out.reshape(user_input.shape[0])
